# Initial kernel scaffold; baseline (speedup 1.0000x reference)
import jax
import jax.numpy as jnp
from jax.experimental import pallas as pl


def kernel(x_q, edge_index_q, x_c, edge_index_c, conv_W, conv_b, att_W, ntn_a_W, ntn_b_W, ntn_bias, fc1_W, fc1_b, fc2_W, fc2_b):
    raise NotImplementedError("write your pallas kernel here")


# speedup vs baseline: 5.5003x; 5.5003x over previous
import functools

import jax
import jax.numpy as jnp
from jax import lax
from jax.experimental import pallas as pl
from jax.experimental.pallas import tpu as pltpu
from jax.experimental.pallas import tpu_sc as plsc

B = 50
N_PER = 200
N = B * N_PER
E = 160000
D = 256
H = D // 2
L = 3
T = 16

NSUB = 16
CHUNK = 128
NCHUNK = 80
EPAD = NSUB * NCHUNK * CHUNK
TRASH = N
NP = 10240
ROWS_PER_SUB = NP // NSUB
RT = 512

_F32 = jnp.float32


def _sc_mesh():
    return plsc.VectorSubcoreMesh(core_axis_name="c", subcore_axis_name="s",
                                  num_cores=2, num_subcores=NSUB)


def _sc_degree(dst_q, dst_c, ones_rows, zeros_rows):
    @functools.partial(
        pl.kernel,
        mesh=_sc_mesh(),
        out_type=[jax.ShapeDtypeStruct((NP, H), _F32)] * 2,
        scratch_types=[
            pltpu.VMEM_SHARED((NP, H), _F32),
            pltpu.VMEM((NCHUNK, CHUNK), jnp.int32),
            pltpu.VMEM((CHUNK, H), _F32),
        ],
    )
    def run(dq_h, dc_h, ones_h, zeros_h, degq_h, degc_h, hist_sh, idx_v, ones_v):
        cid = lax.axis_index("c")
        sid = lax.axis_index("s")

        def branch(d_h, deg_h):
            pltpu.sync_copy(d_h.at[sid], idx_v)
            pltpu.sync_copy(ones_h, ones_v)
            pltpu.sync_copy(zeros_h, hist_sh.at[pl.ds(sid * ROWS_PER_SUB, ROWS_PER_SUB)])
            plsc.subcore_barrier()

            def body(j, carry):
                pltpu.sync_copy(ones_v, hist_sh.at[idx_v.at[j]], add=True)
                return carry

            lax.fori_loop(0, NCHUNK, body, 0)
            plsc.subcore_barrier()
            pltpu.sync_copy(
                hist_sh.at[pl.ds(sid * ROWS_PER_SUB, ROWS_PER_SUB)],
                deg_h.at[pl.ds(sid * ROWS_PER_SUB, ROWS_PER_SUB)],
            )

        @pl.when(cid == 0)
        def _():
            branch(dq_h, degq_h)

        @pl.when(cid == 1)
        def _():
            branch(dc_h, degc_h)

    return run(dst_q, dst_c, ones_rows, zeros_rows)


def _sc_scatter(xws0, xws1, srcs, dsts):
    @functools.partial(
        pl.kernel,
        mesh=_sc_mesh(),
        out_type=[jax.ShapeDtypeStruct((NP, H), _F32)] * 2,
        scratch_types=[
            pltpu.VMEM_SHARED((NP, H), _F32),
            pltpu.VMEM((NCHUNK, CHUNK), jnp.int32),
            pltpu.VMEM((NCHUNK, CHUNK), jnp.int32),
            pltpu.VMEM((CHUNK, H), _F32),
            pltpu.SemaphoreType.DMA,
        ],
    )
    def run(x0_h, x1_h, s_h, d_h, a0_h, a1_h, acc_sh, src_v, dst_v, rows_v, sem):
        cid = lax.axis_index("c")
        sid = lax.axis_index("s")
        pltpu.sync_copy(s_h.at[sid], src_v)
        pltpu.sync_copy(d_h.at[sid], dst_v)

        def branch(x_h, a_h):
            r0 = sid * ROWS_PER_SUB
            pltpu.sync_copy(
                x_h.at[pl.ds(r0, ROWS_PER_SUB)],
                acc_sh.at[pl.ds(r0, ROWS_PER_SUB)],
            )

            plsc.subcore_barrier()

            def body(j, carry):
                pltpu.async_copy(x_h.at[src_v.at[j]], rows_v, sem).wait()
                pltpu.sync_copy(rows_v, acc_sh.at[dst_v.at[j]], add=True)
                return carry

            lax.fori_loop(0, NCHUNK, body, 0)
            plsc.subcore_barrier()
            pltpu.sync_copy(
                acc_sh.at[pl.ds(r0, ROWS_PER_SUB)],
                a_h.at[pl.ds(r0, ROWS_PER_SUB)],
            )

        @pl.when(cid == 0)
        def _():
            branch(x0_h, a0_h)

        @pl.when(cid == 1)
        def _():
            branch(x1_h, a1_h)

    return run(xws0, xws1, srcs, dsts)


def _tc_layer0(x, deg, W):
    def body(x_ref, deg_ref, w_ref, o0_ref, o1_ref):
        dinv = lax.rsqrt(deg_ref[:, :1] + 1.0)
        y = jnp.dot(x_ref[...], w_ref[...], preferred_element_type=_F32) * dinv
        o0_ref[...] = y[:, :H]
        o1_ref[...] = y[:, H:]

    return pl.pallas_call(
        body,
        grid=(NP // RT,),
        in_specs=[
            pl.BlockSpec((RT, D), lambda i: (i, 0)),
            pl.BlockSpec((RT, H), lambda i: (i, 0)),
            pl.BlockSpec((D, D), lambda i: (0, 0)),
        ],
        out_specs=[pl.BlockSpec((RT, H), lambda i: (i, 0))] * 2,
        out_shape=[jax.ShapeDtypeStruct((NP, H), _F32)] * 2,
    )(x, deg, W)


def _tc_layer(a0, a1, deg, W, b_prev):
    def body(a0_ref, a1_ref, deg_ref, w_ref, b_ref, o0_ref, o1_ref):
        dinv = lax.rsqrt(deg_ref[:, :1] + 1.0)
        h = jnp.concatenate([a0_ref[...], a1_ref[...]], axis=1) * dinv + b_ref[...]
        act = jnp.maximum(h, 0.0)
        y = jnp.dot(act, w_ref[...], preferred_element_type=_F32) * dinv
        o0_ref[...] = y[:, :H]
        o1_ref[...] = y[:, H:]

    return pl.pallas_call(
        body,
        grid=(NP // RT,),
        in_specs=[
            pl.BlockSpec((RT, H), lambda i: (i, 0)),
            pl.BlockSpec((RT, H), lambda i: (i, 0)),
            pl.BlockSpec((RT, H), lambda i: (i, 0)),
            pl.BlockSpec((D, D), lambda i: (0, 0)),
            pl.BlockSpec((1, D), lambda i: (0, 0)),
        ],
        out_specs=[pl.BlockSpec((RT, H), lambda i: (i, 0))] * 2,
        out_shape=[jax.ShapeDtypeStruct((NP, H), _F32)] * 2,
    )(a0, a1, deg, W, b_prev)


def _tc_pool(a0, a1, deg, b_last, att_W):
    def body(a0_ref, a1_ref, deg_ref, b_ref, w_ref, e_ref):
        dinv = lax.rsqrt(deg_ref[:, :1] + 1.0)
        h = jnp.concatenate([a0_ref[...], a1_ref[...]], axis=1) * dinv + b_ref[...]
        m = jnp.mean(h, axis=0, keepdims=True)
        ctx = jnp.tanh(jnp.dot(m, w_ref[...], preferred_element_type=_F32))
        sig = jax.nn.sigmoid(jnp.sum(h * ctx, axis=1, keepdims=True))
        e = jnp.sum(h * sig, axis=0, keepdims=True)
        e_ref[...] = jnp.broadcast_to(e, (8, D))

    return pl.pallas_call(
        body,
        grid=(B,),
        in_specs=[
            pl.BlockSpec((N_PER, H), lambda b: (b, 0)),
            pl.BlockSpec((N_PER, H), lambda b: (b, 0)),
            pl.BlockSpec((N_PER, H), lambda b: (b, 0)),
            pl.BlockSpec((1, D), lambda b: (0, 0)),
            pl.BlockSpec((D, D), lambda b: (0, 0)),
        ],
        out_specs=pl.BlockSpec((8, D), lambda b: (b, 0)),
        out_shape=jax.ShapeDtypeStruct((B * 8, D), _F32),
    )(a0, a1, deg, b_last, att_W)


def _tc_ntn(e1, e2, A2, ntn_b_W, ntn_bias, fc1_W, fc1_b, fc2_W, fc2_b):
    def body(e1_ref, e2_ref, a_ref, nb_ref, bias_ref, f1_ref, f1b_ref,
             f2_ref, f2b_ref, out_ref):
        e1v = e1_ref[...]
        e2v = e2_ref[...]
        u = lax.dot_general(e2v, a_ref[...], (((1,), (1,)), ((), ())),
                            preferred_element_type=_F32)
        lane = lax.broadcasted_iota(jnp.int32, (1, T), 1)
        bil = jnp.zeros((B, T), _F32)
        for k in range(T):
            s = jnp.sum(u[:, k * D:(k + 1) * D] * e1v, axis=1, keepdims=True)
            bil = bil + jnp.where(lane == k, s, 0.0)
        cat = jnp.concatenate([e1v, e2v], axis=1)
        lin = jnp.dot(cat, nb_ref[...], preferred_element_type=_F32)
        scores = jnp.maximum(bil + lin + bias_ref[...], 0.0)
        h1 = jnp.maximum(
            jnp.dot(scores, f1_ref[...], preferred_element_type=_F32)
            + f1b_ref[...], 0.0)
        out_ref[...] = jax.nn.sigmoid(
            jnp.dot(h1, f2_ref[...], preferred_element_type=_F32) + f2b_ref[...])

    return pl.pallas_call(
        body,
        out_shape=jax.ShapeDtypeStruct((B, 1), _F32),
    )(e1, e2, A2, ntn_b_W, ntn_bias, fc1_W, fc1_b, fc2_W, fc2_b)


def _pad_edges(ei):
    src = ei[0].astype(jnp.int32)
    dst = ei[1].astype(jnp.int32)
    pad = EPAD - E
    src_p = jnp.concatenate([src, jnp.zeros((pad,), jnp.int32)])
    dst_p = jnp.concatenate([dst, jnp.full((pad,), TRASH, jnp.int32)])
    return src_p.reshape(NSUB, NCHUNK, CHUNK), dst_p.reshape(NSUB, NCHUNK, CHUNK)


def kernel(x_q, edge_index_q, x_c, edge_index_c, conv_W, conv_b, att_W,
           ntn_a_W, ntn_b_W, ntn_bias, fc1_W, fc1_b, fc2_W, fc2_b):
    src_q, dst_q = _pad_edges(edge_index_q)
    src_c, dst_c = _pad_edges(edge_index_c)
    xpad = jnp.zeros((NP - N, D), _F32)
    x_q = jnp.concatenate([x_q.astype(_F32), xpad])
    x_c = jnp.concatenate([x_c.astype(_F32), xpad])

    ones_rows = jnp.ones((CHUNK, H), _F32)
    zeros_rows = jnp.zeros((ROWS_PER_SUB, H), _F32)
    deg_q, deg_c = _sc_degree(dst_q, dst_c, ones_rows, zeros_rows)

    def gnn(x, deg, srcs, dsts):
        xw0, xw1 = _tc_layer0(x, deg, conv_W[0])
        a0, a1 = _sc_scatter(xw0, xw1, srcs, dsts)
        for i in range(1, L):
            xw0, xw1 = _tc_layer(a0, a1, deg, conv_W[i],
                                 conv_b[i - 1].reshape(1, D))
            a0, a1 = _sc_scatter(xw0, xw1, srcs, dsts)
        return a0, a1

    aq0, aq1 = gnn(x_q, deg_q, src_q, dst_q)
    ac0, ac1 = gnn(x_c, deg_c, src_c, dst_c)

    b_last = conv_b[L - 1].reshape(1, D)
    e1 = _tc_pool(aq0, aq1, deg_q, b_last, att_W).reshape(B, 8, D)[:, 0, :]
    e2 = _tc_pool(ac0, ac1, deg_c, b_last, att_W).reshape(B, 8, D)[:, 0, :]

    score = _tc_ntn(
        e1, e2,
        ntn_a_W.reshape(T * D, D),
        ntn_b_W,
        ntn_bias.reshape(1, T),
        fc1_W,
        fc1_b.reshape(1, T),
        fc2_W,
        fc2_b.reshape(1, 1),
    )
    return score[:, 0]


# speedup vs baseline: 6.1897x; 1.1253x over previous
import functools

import jax
import jax.numpy as jnp
from jax import lax
from jax.experimental import pallas as pl
from jax.experimental.pallas import tpu as pltpu
from jax.experimental.pallas import tpu_sc as plsc

B = 50
N_PER = 200
N = B * N_PER
E = 160000
D = 256
H = D // 2
L = 3
T = 16

NSUB = 16
CHUNK = 128
NCHUNK = 80
EPAD = NSUB * NCHUNK * CHUNK
TRASH = N
NBUF = 2
NIDX = 4
NP = 10240
ROWS_PER_SUB = NP // NSUB
RT = 512

_F32 = jnp.float32


def _sc_mesh():
    return plsc.VectorSubcoreMesh(core_axis_name="c", subcore_axis_name="s",
                                  num_cores=2, num_subcores=NSUB)


def _sc_degree(dst_q, dst_c, ones_rows, zeros_rows):
    @functools.partial(
        pl.kernel,
        mesh=_sc_mesh(),
        out_type=[jax.ShapeDtypeStruct((NP, H), _F32)] * 2,
        scratch_types=[
            pltpu.VMEM_SHARED((NP, H), _F32),
            pltpu.VMEM((NCHUNK, CHUNK), jnp.int32),
            pltpu.VMEM((CHUNK, H), _F32),
            pltpu.SemaphoreType.DMA,
        ],
    )
    def run(dq_h, dc_h, ones_h, zeros_h, degq_h, degc_h, hist_sh, idx_v, ones_v,
            sem_s):
        cid = lax.axis_index("c")
        sid = lax.axis_index("s")

        def branch(d_h, deg_h):
            pltpu.sync_copy(d_h.at[sid], idx_v)
            pltpu.sync_copy(ones_h, ones_v)
            pltpu.sync_copy(zeros_h, hist_sh.at[pl.ds(sid * ROWS_PER_SUB, ROWS_PER_SUB)])
            plsc.subcore_barrier()

            def body(j, carry):
                pltpu.async_copy(ones_v, hist_sh.at[idx_v.at[j]], sem_s, add=True)
                return carry

            lax.fori_loop(0, NCHUNK, body, 0)

            def drain(j, carry):
                pltpu.make_async_copy(ones_v, hist_sh.at[idx_v.at[j]], sem_s).wait()
                return carry

            lax.fori_loop(0, NCHUNK, drain, 0)
            plsc.subcore_barrier()
            pltpu.sync_copy(
                hist_sh.at[pl.ds(sid * ROWS_PER_SUB, ROWS_PER_SUB)],
                deg_h.at[pl.ds(sid * ROWS_PER_SUB, ROWS_PER_SUB)],
            )

        @pl.when(cid == 0)
        def _():
            branch(dq_h, degq_h)

        @pl.when(cid == 1)
        def _():
            branch(dc_h, degc_h)

    return run(dst_q, dst_c, ones_rows, zeros_rows)


def _sc_scatter(xws0, xws1, srcs, dsts):
    @functools.partial(
        pl.kernel,
        mesh=_sc_mesh(),
        out_type=[jax.ShapeDtypeStruct((NP, H), _F32)] * 2,
        scratch_types=[
            pltpu.VMEM_SHARED((NP, H), _F32),
            pltpu.VMEM((NIDX, CHUNK), jnp.int32),
            pltpu.VMEM((NCHUNK, CHUNK), jnp.int32),
            pltpu.VMEM((NBUF, CHUNK, H), _F32),
            pltpu.SemaphoreType.DMA((NIDX,)),
            pltpu.SemaphoreType.DMA((NBUF,)),
            pltpu.SemaphoreType.DMA((NBUF,)),
        ],
    )
    def run(x0_h, x1_h, s_h, d_h, a0_h, a1_h, acc_sh, sidx_v, dst_v, rows_v,
            sem_i, sem_g, sem_s):
        cid = lax.axis_index("c")
        sid = lax.axis_index("s")
        pltpu.sync_copy(d_h.at[sid], dst_v)

        def branch(x_h, a_h):
            r0 = sid * ROWS_PER_SUB
            pltpu.sync_copy(
                x_h.at[pl.ds(r0, ROWS_PER_SUB)],
                acc_sh.at[pl.ds(r0, ROWS_PER_SUB)],
            )

            plsc.subcore_barrier()

            def i_issue(j, bi):
                pltpu.async_copy(s_h.at[sid, j], sidx_v.at[bi], sem_i.at[bi])

            def i_wait(j, bi):
                pltpu.make_async_copy(
                    s_h.at[sid, j], sidx_v.at[bi], sem_i.at[bi]).wait()

            def g_issue(j, bi, b):
                pltpu.async_copy(x_h.at[sidx_v.at[bi]], rows_v.at[b],
                                 sem_g.at[b])

            def g_wait(j, bi, b):
                pltpu.make_async_copy(
                    x_h.at[sidx_v.at[bi]], rows_v.at[b], sem_g.at[b]).wait()

            def s_issue(j, b):
                pltpu.async_copy(rows_v.at[b], acc_sh.at[dst_v.at[j]],
                                 sem_s.at[b], add=True)

            def s_wait(j, b):
                pltpu.make_async_copy(
                    rows_v.at[b], acc_sh.at[dst_v.at[j]], sem_s.at[b]).wait()

            for jp in range(NIDX - 1):
                i_issue(jp, jp)
            i_wait(0, 0)
            g_issue(0, 0, 0)

            def step(j, u):
                br = u % NBUF
                bp = (u + 1) % NBUF
                g_wait(j, u % NIDX, br)
                s_issue(j, br)

                @pl.when(j + NIDX - 1 < NCHUNK)
                def _():
                    i_issue(j + NIDX - 1, (u + NIDX - 1) % NIDX)

                @pl.when(j >= 1)
                def _():
                    s_wait(j - 1, bp)

                @pl.when(j + 1 < NCHUNK)
                def _():
                    i_wait(j + 1, (u + 1) % NIDX)
                    g_issue(j + 1, (u + 1) % NIDX, bp)

            def body(jo, carry):
                for u in range(NIDX):
                    step(jo * NIDX + u, u)
                return carry

            lax.fori_loop(0, NCHUNK // NIDX, body, 0)
            s_wait(NCHUNK - 1, (NCHUNK - 1) % NBUF)
            plsc.subcore_barrier()
            pltpu.sync_copy(
                acc_sh.at[pl.ds(r0, ROWS_PER_SUB)],
                a_h.at[pl.ds(r0, ROWS_PER_SUB)],
            )

        @pl.when(cid == 0)
        def _():
            branch(x0_h, a0_h)

        @pl.when(cid == 1)
        def _():
            branch(x1_h, a1_h)

    return run(xws0, xws1, srcs, dsts)


def _tc_layer0(x, deg, W):
    def body(x_ref, deg_ref, w_ref, o0_ref, o1_ref):
        dinv = lax.rsqrt(deg_ref[:, :1] + 1.0)
        y = jnp.dot(x_ref[...], w_ref[...], preferred_element_type=_F32) * dinv
        o0_ref[...] = y[:, :H]
        o1_ref[...] = y[:, H:]

    return pl.pallas_call(
        body,
        grid=(NP // RT,),
        in_specs=[
            pl.BlockSpec((RT, D), lambda i: (i, 0)),
            pl.BlockSpec((RT, H), lambda i: (i, 0)),
            pl.BlockSpec((D, D), lambda i: (0, 0)),
        ],
        out_specs=[pl.BlockSpec((RT, H), lambda i: (i, 0))] * 2,
        out_shape=[jax.ShapeDtypeStruct((NP, H), _F32)] * 2,
    )(x, deg, W)


def _tc_layer(a0, a1, deg, W, b_prev):
    def body(a0_ref, a1_ref, deg_ref, w_ref, b_ref, o0_ref, o1_ref):
        dinv = lax.rsqrt(deg_ref[:, :1] + 1.0)
        h = jnp.concatenate([a0_ref[...], a1_ref[...]], axis=1) * dinv + b_ref[...]
        act = jnp.maximum(h, 0.0)
        y = jnp.dot(act, w_ref[...], preferred_element_type=_F32) * dinv
        o0_ref[...] = y[:, :H]
        o1_ref[...] = y[:, H:]

    return pl.pallas_call(
        body,
        grid=(NP // RT,),
        in_specs=[
            pl.BlockSpec((RT, H), lambda i: (i, 0)),
            pl.BlockSpec((RT, H), lambda i: (i, 0)),
            pl.BlockSpec((RT, H), lambda i: (i, 0)),
            pl.BlockSpec((D, D), lambda i: (0, 0)),
            pl.BlockSpec((1, D), lambda i: (0, 0)),
        ],
        out_specs=[pl.BlockSpec((RT, H), lambda i: (i, 0))] * 2,
        out_shape=[jax.ShapeDtypeStruct((NP, H), _F32)] * 2,
    )(a0, a1, deg, W, b_prev)


def _tc_pool(a0, a1, deg, b_last, att_W):
    def body(a0_ref, a1_ref, deg_ref, b_ref, w_ref, e_ref):
        dinv = lax.rsqrt(deg_ref[:, :1] + 1.0)
        h = jnp.concatenate([a0_ref[...], a1_ref[...]], axis=1) * dinv + b_ref[...]
        m = jnp.mean(h, axis=0, keepdims=True)
        ctx = jnp.tanh(jnp.dot(m, w_ref[...], preferred_element_type=_F32))
        sig = jax.nn.sigmoid(jnp.sum(h * ctx, axis=1, keepdims=True))
        e = jnp.sum(h * sig, axis=0, keepdims=True)
        e_ref[...] = jnp.broadcast_to(e, (8, D))

    return pl.pallas_call(
        body,
        grid=(B,),
        in_specs=[
            pl.BlockSpec((N_PER, H), lambda b: (b, 0)),
            pl.BlockSpec((N_PER, H), lambda b: (b, 0)),
            pl.BlockSpec((N_PER, H), lambda b: (b, 0)),
            pl.BlockSpec((1, D), lambda b: (0, 0)),
            pl.BlockSpec((D, D), lambda b: (0, 0)),
        ],
        out_specs=pl.BlockSpec((8, D), lambda b: (b, 0)),
        out_shape=jax.ShapeDtypeStruct((B * 8, D), _F32),
    )(a0, a1, deg, b_last, att_W)


def _tc_ntn(e1, e2, A2, ntn_b_W, ntn_bias, fc1_W, fc1_b, fc2_W, fc2_b):
    def body(e1_ref, e2_ref, a_ref, nb_ref, bias_ref, f1_ref, f1b_ref,
             f2_ref, f2b_ref, out_ref):
        e1v = e1_ref[...]
        e2v = e2_ref[...]
        u = lax.dot_general(e2v, a_ref[...], (((1,), (1,)), ((), ())),
                            preferred_element_type=_F32)
        lane = lax.broadcasted_iota(jnp.int32, (1, T), 1)
        bil = jnp.zeros((B, T), _F32)
        for k in range(T):
            s = jnp.sum(u[:, k * D:(k + 1) * D] * e1v, axis=1, keepdims=True)
            bil = bil + jnp.where(lane == k, s, 0.0)
        cat = jnp.concatenate([e1v, e2v], axis=1)
        lin = jnp.dot(cat, nb_ref[...], preferred_element_type=_F32)
        scores = jnp.maximum(bil + lin + bias_ref[...], 0.0)
        h1 = jnp.maximum(
            jnp.dot(scores, f1_ref[...], preferred_element_type=_F32)
            + f1b_ref[...], 0.0)
        out_ref[...] = jax.nn.sigmoid(
            jnp.dot(h1, f2_ref[...], preferred_element_type=_F32) + f2b_ref[...])

    return pl.pallas_call(
        body,
        out_shape=jax.ShapeDtypeStruct((B, 1), _F32),
    )(e1, e2, A2, ntn_b_W, ntn_bias, fc1_W, fc1_b, fc2_W, fc2_b)


def _pad_edges(ei):
    src = ei[0].astype(jnp.int32)
    dst = ei[1].astype(jnp.int32)
    pad = EPAD - E
    src_p = jnp.concatenate([src, jnp.zeros((pad,), jnp.int32)])
    dst_p = jnp.concatenate([dst, jnp.full((pad,), TRASH, jnp.int32)])
    return src_p.reshape(NSUB, NCHUNK, CHUNK), dst_p.reshape(NSUB, NCHUNK, CHUNK)


def kernel(x_q, edge_index_q, x_c, edge_index_c, conv_W, conv_b, att_W,
           ntn_a_W, ntn_b_W, ntn_bias, fc1_W, fc1_b, fc2_W, fc2_b):
    src_q, dst_q = _pad_edges(edge_index_q)
    src_c, dst_c = _pad_edges(edge_index_c)
    xpad = jnp.zeros((NP - N, D), _F32)
    x_q = jnp.concatenate([x_q.astype(_F32), xpad])
    x_c = jnp.concatenate([x_c.astype(_F32), xpad])

    ones_rows = jnp.ones((CHUNK, H), _F32)
    zeros_rows = jnp.zeros((ROWS_PER_SUB, H), _F32)
    deg_q, deg_c = _sc_degree(dst_q, dst_c, ones_rows, zeros_rows)

    def gnn(x, deg, srcs, dsts):
        xw0, xw1 = _tc_layer0(x, deg, conv_W[0])
        a0, a1 = _sc_scatter(xw0, xw1, srcs, dsts)
        for i in range(1, L):
            xw0, xw1 = _tc_layer(a0, a1, deg, conv_W[i],
                                 conv_b[i - 1].reshape(1, D))
            a0, a1 = _sc_scatter(xw0, xw1, srcs, dsts)
        return a0, a1

    aq0, aq1 = gnn(x_q, deg_q, src_q, dst_q)
    ac0, ac1 = gnn(x_c, deg_c, src_c, dst_c)

    b_last = conv_b[L - 1].reshape(1, D)
    e1 = _tc_pool(aq0, aq1, deg_q, b_last, att_W).reshape(B, 8, D)[:, 0, :]
    e2 = _tc_pool(ac0, ac1, deg_c, b_last, att_W).reshape(B, 8, D)[:, 0, :]

    score = _tc_ntn(
        e1, e2,
        ntn_a_W.reshape(T * D, D),
        ntn_b_W,
        ntn_bias.reshape(1, T),
        fc1_W,
        fc1_b.reshape(1, T),
        fc2_W,
        fc2_b.reshape(1, 1),
    )
    return score[:, 0]


# speedup vs baseline: 6.7470x; 1.0900x over previous
import functools

import jax
import jax.numpy as jnp
from jax import lax
from jax.experimental import pallas as pl
from jax.experimental.pallas import tpu as pltpu
from jax.experimental.pallas import tpu_sc as plsc

B = 50
N_PER = 200
N = B * N_PER
E = 160000
D = 256
H = D // 2
L = 3
T = 16

NSUB = 16
CHUNK = 128
NCHUNK = 80
EPAD = NSUB * NCHUNK * CHUNK
TRASH = N
NBUF = 2
NIDX = 4
NP = 10240
ROWS_PER_SUB = NP // NSUB
RT = 512

_F32 = jnp.float32


def _sc_mesh():
    return plsc.VectorSubcoreMesh(core_axis_name="c", subcore_axis_name="s",
                                  num_cores=2, num_subcores=NSUB)


def _sc_degree(dsts2, ones_rows, zeros_rows):
    @functools.partial(
        pl.kernel,
        mesh=_sc_mesh(),
        out_type=jax.ShapeDtypeStruct((2, NP, H), _F32),
        scratch_types=[
            pltpu.VMEM_SHARED((NP, H), _F32),
            pltpu.VMEM((NCHUNK, CHUNK), jnp.int32),
            pltpu.VMEM((CHUNK, H), _F32),
            pltpu.SemaphoreType.DMA,
        ],
    )
    def run(d_h, ones_h, zeros_h, deg_h, hist_sh, idx_v, ones_v, sem_s):
        cid = lax.axis_index("c")
        sid = lax.axis_index("s")
        r0 = sid * ROWS_PER_SUB
        pltpu.sync_copy(d_h.at[cid, sid], idx_v)
        pltpu.sync_copy(ones_h, ones_v)
        pltpu.sync_copy(zeros_h, hist_sh.at[pl.ds(r0, ROWS_PER_SUB)])
        plsc.subcore_barrier()

        def body(j, carry):
            pltpu.async_copy(ones_v, hist_sh.at[idx_v.at[j]], sem_s, add=True)
            return carry

        lax.fori_loop(0, NCHUNK, body, 0)

        def drain(j, carry):
            pltpu.make_async_copy(ones_v, hist_sh.at[idx_v.at[j]], sem_s).wait()
            return carry

        lax.fori_loop(0, NCHUNK, drain, 0)
        plsc.subcore_barrier()
        pltpu.sync_copy(
            hist_sh.at[pl.ds(r0, ROWS_PER_SUB)],
            deg_h.at[cid, pl.ds(r0, ROWS_PER_SUB)],
        )

    return run(dsts2, ones_rows, zeros_rows)


def _sc_scatter(xw2, pairs):
    @functools.partial(
        pl.kernel,
        mesh=_sc_mesh(),
        out_type=jax.ShapeDtypeStruct((2, NP, H), _F32),
        scratch_types=[
            pltpu.VMEM_SHARED((NP, H), _F32),
            pltpu.VMEM((NIDX, 2, CHUNK), jnp.int32),
            pltpu.VMEM((NBUF, CHUNK, H), _F32),
            pltpu.SemaphoreType.DMA((NIDX,)),
            pltpu.SemaphoreType.DMA((NBUF,)),
            pltpu.SemaphoreType.DMA((NBUF,)),
        ],
    )
    def run(x_h, p_h, a_h, acc_sh, idx_v, rows_v, sem_i, sem_g, sem_s):
        cid = lax.axis_index("c")
        sid = lax.axis_index("s")
        r0 = sid * ROWS_PER_SUB
        xc_h = x_h.at[cid]
        pltpu.sync_copy(
            xc_h.at[pl.ds(r0, ROWS_PER_SUB)],
            acc_sh.at[pl.ds(r0, ROWS_PER_SUB)],
        )
        plsc.subcore_barrier()

        def i_issue(j, bi):
            pltpu.async_copy(p_h.at[sid, j], idx_v.at[bi], sem_i.at[bi])

        def i_wait(j, bi):
            pltpu.make_async_copy(
                p_h.at[sid, j], idx_v.at[bi], sem_i.at[bi]).wait()

        def g_issue(bi, b):
            pltpu.async_copy(xc_h.at[idx_v.at[bi, 0]], rows_v.at[b],
                             sem_g.at[b])

        def g_wait(bi, b):
            pltpu.make_async_copy(
                xc_h.at[idx_v.at[bi, 0]], rows_v.at[b], sem_g.at[b]).wait()

        def s_issue(bi, b):
            pltpu.async_copy(rows_v.at[b], acc_sh.at[idx_v.at[bi, 1]],
                             sem_s.at[b], add=True)

        def s_wait(bi, b):
            pltpu.make_async_copy(
                rows_v.at[b], acc_sh.at[idx_v.at[bi, 1]], sem_s.at[b]).wait()

        for jp in range(NIDX - 1):
            i_issue(jp, jp)
        i_wait(0, 0)
        g_issue(0, 0)

        def step(j, u):
            br = u % NBUF
            bp = (u + 1) % NBUF
            g_wait(u % NIDX, br)
            s_issue(u % NIDX, br)

            @pl.when(j >= 1)
            def _():
                s_wait((u + NIDX - 1) % NIDX, bp)

            @pl.when(j + NIDX - 1 < NCHUNK)
            def _():
                i_issue(j + NIDX - 1, (u + NIDX - 1) % NIDX)

            @pl.when(j + 1 < NCHUNK)
            def _():
                i_wait(j + 1, (u + 1) % NIDX)
                g_issue((u + 1) % NIDX, bp)

        def body(jo, carry):
            for u in range(NIDX):
                step(jo * NIDX + u, u)
            return carry

        lax.fori_loop(0, NCHUNK // NIDX, body, 0)
        s_wait((NCHUNK - 1) % NIDX, (NCHUNK - 1) % NBUF)
        plsc.subcore_barrier()
        pltpu.sync_copy(
            acc_sh.at[pl.ds(r0, ROWS_PER_SUB)],
            a_h.at[cid, pl.ds(r0, ROWS_PER_SUB)],
        )

    return run(xw2, pairs)


def _tc_layer0(x, deg2, W, bq):
    def body(x_ref, deg_ref, w_ref, o_ref):
        dinv = lax.rsqrt(deg_ref[0][:, :1] + 1.0)
        y = jnp.dot(x_ref[...], w_ref[...], preferred_element_type=_F32) * dinv
        o_ref[0] = y[:, :H]
        o_ref[1] = y[:, H:]

    return pl.pallas_call(
        body,
        grid=(NP // RT,),
        in_specs=[
            pl.BlockSpec((RT, D), lambda i: (i, 0)),
            pl.BlockSpec((1, RT, H), lambda i: (bq, i, 0)),
            pl.BlockSpec((D, D), lambda i: (0, 0)),
        ],
        out_specs=pl.BlockSpec((2, RT, H), lambda i: (0, i, 0)),
        out_shape=jax.ShapeDtypeStruct((2, NP, H), _F32),
    )(x, deg2, W)


def _tc_layer(a2, deg2, W, b_prev, bq):
    def body(a_ref, deg_ref, w_ref, b_ref, o_ref):
        dinv = lax.rsqrt(deg_ref[0][:, :1] + 1.0)
        h = jnp.concatenate([a_ref[0], a_ref[1]], axis=1) * dinv + b_ref[...]
        act = jnp.maximum(h, 0.0)
        y = jnp.dot(act, w_ref[...], preferred_element_type=_F32) * dinv
        o_ref[0] = y[:, :H]
        o_ref[1] = y[:, H:]

    return pl.pallas_call(
        body,
        grid=(NP // RT,),
        in_specs=[
            pl.BlockSpec((2, RT, H), lambda i: (0, i, 0)),
            pl.BlockSpec((1, RT, H), lambda i: (bq, i, 0)),
            pl.BlockSpec((D, D), lambda i: (0, 0)),
            pl.BlockSpec((1, D), lambda i: (0, 0)),
        ],
        out_specs=pl.BlockSpec((2, RT, H), lambda i: (0, i, 0)),
        out_shape=jax.ShapeDtypeStruct((2, NP, H), _F32),
    )(a2, deg2, W, b_prev)


def _tc_pool(a2, deg2, b_last, att_W, bq):
    def body(a_ref, deg_ref, b_ref, w_ref, e_ref):
        dinv = lax.rsqrt(deg_ref[0][:, :1] + 1.0)
        h = jnp.concatenate([a_ref[0], a_ref[1]], axis=1) * dinv + b_ref[...]
        m = jnp.mean(h, axis=0, keepdims=True)
        ctx = jnp.tanh(jnp.dot(m, w_ref[...], preferred_element_type=_F32))
        sig = jax.nn.sigmoid(jnp.sum(h * ctx, axis=1, keepdims=True))
        e = jnp.sum(h * sig, axis=0, keepdims=True)
        e_ref[...] = jnp.broadcast_to(e, (8, D))

    return pl.pallas_call(
        body,
        grid=(B,),
        in_specs=[
            pl.BlockSpec((2, N_PER, H), lambda b: (0, b, 0)),
            pl.BlockSpec((1, N_PER, H), lambda b: (bq, b, 0)),
            pl.BlockSpec((1, D), lambda b: (0, 0)),
            pl.BlockSpec((D, D), lambda b: (0, 0)),
        ],
        out_specs=pl.BlockSpec((8, D), lambda b: (b, 0)),
        out_shape=jax.ShapeDtypeStruct((B * 8, D), _F32),
    )(a2, deg2, b_last, att_W)


def _tc_ntn(e1, e2, A2, ntn_b_W, ntn_bias, fc1_W, fc1_b, fc2_W, fc2_b):
    def body(e1_ref, e2_ref, a_ref, nb_ref, bias_ref, f1_ref, f1b_ref,
             f2_ref, f2b_ref, out_ref):
        e1v = e1_ref[...]
        e2v = e2_ref[...]
        u = lax.dot_general(e2v, a_ref[...], (((1,), (1,)), ((), ())),
                            preferred_element_type=_F32)
        lane = lax.broadcasted_iota(jnp.int32, (1, T), 1)
        bil = jnp.zeros((B, T), _F32)
        for k in range(T):
            s = jnp.sum(u[:, k * D:(k + 1) * D] * e1v, axis=1, keepdims=True)
            bil = bil + jnp.where(lane == k, s, 0.0)
        cat = jnp.concatenate([e1v, e2v], axis=1)
        lin = jnp.dot(cat, nb_ref[...], preferred_element_type=_F32)
        scores = jnp.maximum(bil + lin + bias_ref[...], 0.0)
        h1 = jnp.maximum(
            jnp.dot(scores, f1_ref[...], preferred_element_type=_F32)
            + f1b_ref[...], 0.0)
        out_ref[...] = jax.nn.sigmoid(
            jnp.dot(h1, f2_ref[...], preferred_element_type=_F32) + f2b_ref[...])

    return pl.pallas_call(
        body,
        out_shape=jax.ShapeDtypeStruct((B, 1), _F32),
    )(e1, e2, A2, ntn_b_W, ntn_bias, fc1_W, fc1_b, fc2_W, fc2_b)


def _pad_edges(ei):
    src = ei[0].astype(jnp.int32)
    dst = ei[1].astype(jnp.int32)
    pad = EPAD - E
    src_p = jnp.concatenate([src, jnp.zeros((pad,), jnp.int32)])
    dst_p = jnp.concatenate([dst, jnp.full((pad,), TRASH, jnp.int32)])
    pairs = jnp.stack(
        [src_p.reshape(NSUB, NCHUNK, CHUNK), dst_p.reshape(NSUB, NCHUNK, CHUNK)],
        axis=2)
    return pairs, dst_p.reshape(NSUB, NCHUNK, CHUNK)


def kernel(x_q, edge_index_q, x_c, edge_index_c, conv_W, conv_b, att_W,
           ntn_a_W, ntn_b_W, ntn_bias, fc1_W, fc1_b, fc2_W, fc2_b):
    pairs_q, dst_q = _pad_edges(edge_index_q)
    pairs_c, dst_c = _pad_edges(edge_index_c)
    xpad = jnp.zeros((NP - N, D), _F32)
    x_q = jnp.concatenate([x_q.astype(_F32), xpad])
    x_c = jnp.concatenate([x_c.astype(_F32), xpad])

    ones_rows = jnp.ones((CHUNK, H), _F32)
    zeros_rows = jnp.zeros((ROWS_PER_SUB, H), _F32)
    deg2 = _sc_degree(jnp.stack([dst_q, dst_c]), ones_rows, zeros_rows)

    def gnn(x, pairs, bq):
        xw = _tc_layer0(x, deg2, conv_W[0], bq)
        a2 = _sc_scatter(xw, pairs)
        for i in range(1, L):
            xw = _tc_layer(a2, deg2, conv_W[i], conv_b[i - 1].reshape(1, D), bq)
            a2 = _sc_scatter(xw, pairs)
        return a2

    aq = gnn(x_q, pairs_q, 0)
    ac = gnn(x_c, pairs_c, 1)

    b_last = conv_b[L - 1].reshape(1, D)
    e1 = _tc_pool(aq, deg2, b_last, att_W, 0).reshape(B, 8, D)[:, 0, :]
    e2 = _tc_pool(ac, deg2, b_last, att_W, 1).reshape(B, 8, D)[:, 0, :]

    score = _tc_ntn(
        e1, e2,
        ntn_a_W.reshape(T * D, D),
        ntn_b_W,
        ntn_bias.reshape(1, T),
        fc1_W,
        fc1_b.reshape(1, T),
        fc2_W,
        fc2_b.reshape(1, 1),
    )
    return score[:, 0]
